# Initial kernel scaffold; baseline (speedup 1.0000x reference)
import jax
import jax.numpy as jnp
from jax.experimental import pallas as pl


def kernel(x, adj, W1a, b1a, W1b, b1b, W2a, b2a, W2b, b2b, W3a, b3a, W3b, b3b):
    raise NotImplementedError("write your pallas kernel here")


# speedup vs baseline: 2265.3478x; 2265.3478x over previous
import jax
import jax.numpy as jnp
from jax.experimental import pallas as pl
from jax.experimental.pallas import tpu as pltpu


def _split(v):
    hi = v.astype(jnp.bfloat16)
    lo = (v - hi.astype(jnp.float32)).astype(jnp.bfloat16)
    return hi, lo


def _dot(a, b):
    return jax.lax.dot_general(
        a, b, (((1,), (0,)), ((), ())), preferred_element_type=jnp.float32
    )


def _net_kernel(xT_ref, adj_ref, W1aT_ref, b1a_ref, W1bT_ref, b1b_ref,
                W2aT_ref, b2a_ref, W2bT_ref, b2b_ref,
                W3aT_ref, b3a_ref, W3bT_ref, b3b_ref, out_ref):
    A = adj_ref[...].astype(jnp.bfloat16)
    g = xT_ref[...]

    def agg_dot(t):
        hi, lo = _split(t)
        return _dot(hi, A) + _dot(lo, A)

    def mlp_dot(wT, t):
        w1, w2 = _split(wT)
        t1, t2 = _split(t)
        return _dot(w1, t1) + (_dot(w1, t2) + _dot(w2, t1))

    def gin_layer(g, WaT_ref, ba_ref, WbT_ref, bb_ref):
        z = g + agg_dot(g)
        u = jnp.maximum(mlp_dot(WaT_ref[...], z) + ba_ref[...], 0.0)
        return mlp_dot(WbT_ref[...], u) + bb_ref[...]

    g = jnp.maximum(gin_layer(g, W1aT_ref, b1a_ref, W1bT_ref, b1b_ref), 0.0)
    g = jnp.maximum(gin_layer(g, W2aT_ref, b2a_ref, W2bT_ref, b2b_ref), 0.0)
    g = gin_layer(g, W3aT_ref, b3a_ref, W3bT_ref, b3b_ref)
    out_ref[...] = jnp.mean(g, axis=1, keepdims=True)


@jax.jit
def _run(xT, adj, *wb):
    O = wb[8].shape[0]
    out = pl.pallas_call(
        _net_kernel,
        out_shape=jax.ShapeDtypeStruct((O, 1), jnp.float32),
        compiler_params=pltpu.CompilerParams(
            vmem_limit_bytes=100 * 1024 * 1024,
        ),
    )(xT, adj, *wb)
    return out.reshape(1, O)


def kernel(x, adj, W1a, b1a, W1b, b1b, W2a, b2a, W2b, b2b, W3a, b3a, W3b, b3b):
    col = lambda b: b.reshape(-1, 1)
    return _run(
        x.T, adj,
        W1a.T, col(b1a), W1b.T, col(b1b),
        W2a.T, col(b2a), W2b.T, col(b2b),
        W3a.T, col(b3a), W3b.T, col(b3b),
    )


# speedup vs baseline: 5219.4115x; 2.3040x over previous
import jax
import jax.numpy as jnp
from jax.experimental import pallas as pl
from jax.experimental.pallas import tpu as pltpu


def _split(v):
    hi = v.astype(jnp.bfloat16)
    lo = (v - hi.astype(jnp.float32)).astype(jnp.bfloat16)
    return hi, lo


def _dot(a, b):
    return jax.lax.dot_general(
        a, b, (((1,), (0,)), ((), ())), preferred_element_type=jnp.float32
    )


def _net_kernel(x_ref, adj_ref, W1a_ref, b1a_ref, W1b_ref, b1b_ref,
                W2a_ref, b2a_ref, W2b_ref, b2b_ref,
                W3a_ref, b3a_ref, W3b_ref, b3b_ref, out_ref):
    A = adj_ref[...].astype(jnp.bfloat16)
    g = x_ref[...].T

    def agg_dot(t):
        hi, lo = _split(t)
        return _dot(hi, A) + _dot(lo, A)

    def mlp_dot(w_ref, t):
        w1, w2 = _split(w_ref[...].T)
        t1, t2 = _split(t)
        return _dot(w1, t1) + (_dot(w1, t2) + _dot(w2, t1))

    def gin_layer(g, Wa_ref, ba_ref, Wb_ref, bb_ref):
        z = g + agg_dot(g)
        u = jnp.maximum(mlp_dot(Wa_ref, z) + ba_ref[...].reshape(-1, 1), 0.0)
        return mlp_dot(Wb_ref, u) + bb_ref[...].reshape(-1, 1)

    g = jnp.maximum(gin_layer(g, W1a_ref, b1a_ref, W1b_ref, b1b_ref), 0.0)
    g = jnp.maximum(gin_layer(g, W2a_ref, b2a_ref, W2b_ref, b2b_ref), 0.0)
    g = gin_layer(g, W3a_ref, b3a_ref, W3b_ref, b3b_ref)
    out_ref[...] = jnp.mean(g, axis=1, keepdims=True).T


@jax.jit
def kernel(x, adj, W1a, b1a, W1b, b1b, W2a, b2a, W2b, b2b, W3a, b3a, W3b, b3b):
    O = W3b.shape[1]
    return pl.pallas_call(
        _net_kernel,
        out_shape=jax.ShapeDtypeStruct((1, O), jnp.float32),
        compiler_params=pltpu.CompilerParams(
            vmem_limit_bytes=100 * 1024 * 1024,
        ),
    )(x, adj, W1a, b1a, W1b, b1b, W2a, b2a, W2b, b2b, W3a, b3a, W3b, b3b)
